# baseline (speedup 1.0000x reference)
import functools
import jax
import jax.numpy as jnp
from jax import lax
from jax.experimental import pallas as pl
from jax.experimental.pallas import tpu as pltpu

M, B, D, C = 8, 16384, 128, 10


def _tc_body(x_ref, w_ref, b_ref, out_ref):
    bs = x_ref.shape[0]
    logits = jnp.dot(x_ref[:], w_ref[:], preferred_element_type=jnp.float32)
    logits = logits + b_ref[:]
    iota = lax.broadcasted_iota(jnp.int32, (bs, C), 1)
    acc = jnp.zeros((bs, C), jnp.float32)
    for m in range(M):
        lm = logits[:, m * C:(m + 1) * C]
        mx = jnp.max(lm, axis=1, keepdims=True)
        cand = jnp.where(lm >= mx, iota, C)
        amin = jnp.min(cand, axis=1, keepdims=True)
        acc = acc + (iota == amin).astype(jnp.float32)
    out_ref[:] = acc


def kernel(x, W, b):
    W2 = jnp.transpose(W, (1, 0, 2)).reshape(D, M * C)
    b2 = b.reshape(1, M * C)
    bs = 2048
    grid = (B // bs,)
    out = pl.pallas_call(
        _tc_body,
        grid=grid,
        in_specs=[
            pl.BlockSpec((bs, D), lambda i: (i, 0)),
            pl.BlockSpec((D, M * C), lambda i: (0, 0)),
            pl.BlockSpec((1, M * C), lambda i: (0, 0)),
        ],
        out_specs=pl.BlockSpec((bs, C), lambda i: (i, 0)),
        out_shape=jax.ShapeDtypeStruct((B, C), jnp.float32),
    )(x, W2, b2)
    return out


# speedup vs baseline: 4.2003x; 4.2003x over previous
import functools
import jax
import jax.numpy as jnp
from jax import lax
from jax.experimental import pallas as pl
from jax.experimental.pallas import tpu as pltpu

M, B, D, C = 8, 16384, 128, 10
CP = 16
NEG = jnp.float32(-3.0e38)


def _tc_body(x_ref, w_ref, b_ref, out_ref):
    bs = x_ref.shape[0]
    xT = x_ref[:].T
    logitsT = jnp.dot(w_ref[:], xT, preferred_element_type=jnp.float32)
    logitsT = logitsT + b_ref[:]
    l3 = logitsT.reshape(M, CP, bs)
    mx = jnp.max(l3, axis=1, keepdims=True)
    iota = lax.broadcasted_iota(jnp.int32, (M, CP, bs), 1)
    cand = jnp.where(l3 >= mx, iota, CP)
    am = jnp.min(cand, axis=1, keepdims=True)
    oh = (iota == am).astype(jnp.float32)
    countsT = jnp.sum(oh, axis=0)
    out_ref[:] = countsT.T[:, :C]


def kernel(x, W, b):
    W4 = jnp.zeros((M, CP, D), jnp.float32).at[:, :C, :].set(
        jnp.transpose(W, (0, 2, 1))).reshape(M * CP, D)
    b4 = jnp.full((M, CP), NEG, jnp.float32).at[:, :C].set(b)
    b4 = b4.reshape(M * CP, 1)
    bs = 2048
    grid = (B // bs,)
    out = pl.pallas_call(
        _tc_body,
        grid=grid,
        in_specs=[
            pl.BlockSpec((bs, D), lambda i: (i, 0)),
            pl.BlockSpec((M * CP, D), lambda i: (0, 0)),
            pl.BlockSpec((M * CP, 1), lambda i: (0, 0)),
        ],
        out_specs=pl.BlockSpec((bs, C), lambda i: (i, 0)),
        out_shape=jax.ShapeDtypeStruct((B, C), jnp.float32),
    )(x, W4, b4)
    return out
